# initial kernel scaffold (unmeasured)
import jax
import jax.numpy as jnp
from jax import lax
from jax.experimental import pallas as pl
from jax.experimental.pallas import tpu as pltpu


TM = 1024
TN = 1024
TK = 2048


def _mm_body(dy_ref, w_ref, out_ref, acc_ref):
    k = pl.program_id(2)

    @pl.when(k == 0)
    def _():
        acc_ref[...] = jnp.zeros_like(acc_ref)

    a = dy_ref[...].astype(jnp.bfloat16)
    b = w_ref[...].astype(jnp.bfloat16)
    acc_ref[...] += lax.dot_general(
        a, b, (((1,), (1,)), ((), ())), preferred_element_type=jnp.float32
    )

    @pl.when(k == pl.num_programs(2) - 1)
    def _():
        out_ref[...] = acc_ref[...].astype(jnp.bfloat16)


def _local_partial(dy, w):
    m, kdim = dy.shape
    n = w.shape[0]
    grid = (m // TM, n // TN, kdim // TK)
    return pl.pallas_call(
        _mm_body,
        grid=grid,
        in_specs=[
            pl.BlockSpec((TM, TK), lambda i, j, k: (i, k)),
            pl.BlockSpec((TN, TK), lambda i, j, k: (j, k)),
        ],
        out_specs=pl.BlockSpec((TM, TN), lambda i, j, k: (i, j)),
        out_shape=jax.ShapeDtypeStruct((m, n), jnp.bfloat16),
        scratch_shapes=[pltpu.VMEM((TM, TN), jnp.float32)],
    )(dy, w)


CHUNK = 512


def _ar_body(p_ref, out_ref, recv_ref, a_ref, b_ref, o_ref, lsems, send_sem, recv_sem):
    my_x = lax.axis_index("x")
    my_y = lax.axis_index("y")
    my_z = lax.axis_index("z")

    rdma = pltpu.make_async_remote_copy(
        src_ref=p_ref,
        dst_ref=recv_ref,
        send_sem=send_sem,
        recv_sem=recv_sem,
        device_id=(1 - my_x, my_y, my_z),
        device_id_type=pl.DeviceIdType.MESH,
    )
    rdma.start()
    rdma.wait_send()
    rdma.wait_recv()

    n_chunks = p_ref.shape[0] // CHUNK
    for c in range(n_chunks):
        sl = pl.ds(c * CHUNK, CHUNK)
        cp_a = pltpu.make_async_copy(p_ref.at[sl], a_ref, lsems.at[0])
        cp_b = pltpu.make_async_copy(recv_ref.at[sl], b_ref, lsems.at[1])
        cp_a.start()
        cp_b.start()
        cp_a.wait()
        cp_b.wait()
        o_ref[...] = a_ref[...].astype(jnp.float32) + b_ref[...].astype(jnp.float32)
        cp_o = pltpu.make_async_copy(o_ref, out_ref.at[sl], lsems.at[2])
        cp_o.start()
        cp_o.wait()


def _allreduce_x(p):
    m, n = p.shape
    out, _ = pl.pallas_call(
        _ar_body,
        in_specs=[pl.BlockSpec(memory_space=pl.ANY)],
        out_specs=[
            pl.BlockSpec(memory_space=pl.ANY),
            pl.BlockSpec(memory_space=pl.ANY),
        ],
        out_shape=[
            jax.ShapeDtypeStruct((m, n), jnp.float32),
            jax.ShapeDtypeStruct((m, n), jnp.bfloat16),
        ],
        scratch_shapes=[
            pltpu.VMEM((CHUNK, n), jnp.bfloat16),
            pltpu.VMEM((CHUNK, n), jnp.bfloat16),
            pltpu.VMEM((CHUNK, n), jnp.float32),
            pltpu.SemaphoreType.DMA((3,)),
            pltpu.SemaphoreType.DMA,
            pltpu.SemaphoreType.DMA,
        ],
        compiler_params=pltpu.CompilerParams(collective_id=0),
    )(p)
    return out


def kernel(dy, W):
    p = _local_partial(dy, W)
    return _allreduce_x(p)


# baseline (device time: 844734 ns/iter reference)
import jax
import jax.numpy as jnp
from jax import lax
from jax.experimental import pallas as pl
from jax.experimental.pallas import tpu as pltpu


TM = 1024
TN = 1024
TK = 2048


def _mm_body(dy_ref, w_ref, out_ref, acc_ref):
    k = pl.program_id(2)

    @pl.when(k == 0)
    def _():
        acc_ref[...] = jnp.zeros_like(acc_ref)

    a = dy_ref[...].astype(jnp.bfloat16)
    b = w_ref[...].astype(jnp.bfloat16)
    acc_ref[...] += lax.dot_general(
        a, b, (((1,), (1,)), ((), ())), preferred_element_type=jnp.float32
    )

    @pl.when(k == pl.num_programs(2) - 1)
    def _():
        out_ref[...] = acc_ref[...].astype(jnp.bfloat16)


def _local_partial(dy, w):
    m, kdim = dy.shape
    n = w.shape[0]
    grid = (m // TM, n // TN, kdim // TK)
    return pl.pallas_call(
        _mm_body,
        grid=grid,
        in_specs=[
            pl.BlockSpec((TM, TK), lambda i, j, k: (i, k)),
            pl.BlockSpec((TN, TK), lambda i, j, k: (j, k)),
        ],
        out_specs=pl.BlockSpec((TM, TN), lambda i, j, k: (i, j)),
        out_shape=jax.ShapeDtypeStruct((m, n), jnp.bfloat16),
        scratch_shapes=[pltpu.VMEM((TM, TN), jnp.float32)],
        compiler_params=pltpu.CompilerParams(vmem_limit_bytes=100 * 1024 * 1024),
    )(dy, w)


CHUNK = 512


def _ar_body(p_ref, out_ref, recv_ref, a_ref, b_ref, o_ref, lsems, send_sem, recv_sem):
    my_x = lax.axis_index("x")
    my_y = lax.axis_index("y")
    my_z = lax.axis_index("z")

    rdma = pltpu.make_async_remote_copy(
        src_ref=p_ref,
        dst_ref=recv_ref,
        send_sem=send_sem,
        recv_sem=recv_sem,
        device_id=(1 - my_x, my_y, my_z),
        device_id_type=pl.DeviceIdType.MESH,
    )
    rdma.start()
    rdma.wait_send()
    rdma.wait_recv()

    n_chunks = p_ref.shape[0] // CHUNK
    for c in range(n_chunks):
        sl = pl.ds(c * CHUNK, CHUNK)
        cp_a = pltpu.make_async_copy(p_ref.at[sl], a_ref, lsems.at[0])
        cp_b = pltpu.make_async_copy(recv_ref.at[sl], b_ref, lsems.at[1])
        cp_a.start()
        cp_b.start()
        cp_a.wait()
        cp_b.wait()
        o_ref[...] = a_ref[...].astype(jnp.float32) + b_ref[...].astype(jnp.float32)
        cp_o = pltpu.make_async_copy(o_ref, out_ref.at[sl], lsems.at[2])
        cp_o.start()
        cp_o.wait()


def _allreduce_x(p):
    m, n = p.shape
    out, _ = pl.pallas_call(
        _ar_body,
        in_specs=[pl.BlockSpec(memory_space=pl.ANY)],
        out_specs=[
            pl.BlockSpec(memory_space=pl.ANY),
            pl.BlockSpec(memory_space=pl.ANY),
        ],
        out_shape=[
            jax.ShapeDtypeStruct((m, n), jnp.float32),
            jax.ShapeDtypeStruct((m, n), jnp.bfloat16),
        ],
        scratch_shapes=[
            pltpu.VMEM((CHUNK, n), jnp.bfloat16),
            pltpu.VMEM((CHUNK, n), jnp.bfloat16),
            pltpu.VMEM((CHUNK, n), jnp.float32),
            pltpu.SemaphoreType.DMA((3,)),
            pltpu.SemaphoreType.DMA,
            pltpu.SemaphoreType.DMA,
        ],
    )(p)
    return out


def kernel(dy, W):
    p = _local_partial(dy, W)
    return _allreduce_x(p)


# device time: 385688 ns/iter; 2.1902x vs baseline; 2.1902x over previous
import jax
import jax.numpy as jnp
from jax import lax
from jax.experimental import pallas as pl
from jax.experimental.pallas import tpu as pltpu

NY = 4
NZ = 4
BLK = 256
HALF = 2048


TN = 1024
TK = 2048


def _mm_body(dy_ref, w_ref, out_ref, acc_ref):
    k = pl.program_id(1)

    @pl.when(k == 0)
    def _():
        acc_ref[...] = jnp.zeros_like(acc_ref)

    a = dy_ref[:, pl.ds(k * TK, TK)].astype(jnp.bfloat16)
    b = w_ref[...].astype(jnp.bfloat16)
    acc_ref[...] += lax.dot_general(
        a, b, (((1,), (1,)), ((), ())), preferred_element_type=jnp.float32
    )

    @pl.when(k == pl.num_programs(1) - 1)
    def _():
        out_ref[...] = acc_ref[...].astype(jnp.bfloat16)


def _local_partial_block(dy_blk, w):
    m, kdim = dy_blk.shape
    n = w.shape[0]
    grid = (n // TN, kdim // TK)
    return pl.pallas_call(
        _mm_body,
        grid=grid,
        in_specs=[
            pl.BlockSpec((m, kdim), lambda j, k: (0, 0)),
            pl.BlockSpec((TN, TK), lambda j, k: (j, k)),
        ],
        out_specs=pl.BlockSpec((m, TN), lambda j, k: (0, j)),
        out_shape=jax.ShapeDtypeStruct((m, n), jnp.bfloat16),
        scratch_shapes=[pltpu.VMEM((m, TN), jnp.float32)],
        compiler_params=pltpu.CompilerParams(vmem_limit_bytes=100 * 1024 * 1024),
    )(dy_blk, w)


def _comm_body(
    p_ref,
    out_ref,
    ha_ref,
    hb_ref,
    pxr_ref,
    va_ref,
    vb_ref,
    vs_ref,
    aa_ref,
    ab_ref,
    ao_ref,
    lsems,
    sx_sem,
    rx_sem,
    sA1,
    rA1,
    sB1,
    rB1,
    sA2,
    rA2,
    sB2,
    rB2,
):
    x = lax.axis_index("x")
    y = lax.axis_index("y")
    z = lax.axis_index("z")
    gA = y + 4 * z
    gB = z + 4 * y

    def _remote(src, dst, ssem, rsem, dev):
        return pltpu.make_async_remote_copy(
            src_ref=src,
            dst_ref=dst,
            send_sem=ssem,
            recv_sem=rsem,
            device_id=dev,
            device_id_type=pl.DeviceIdType.MESH,
        )

    rdx = _remote(p_ref, pxr_ref, sx_sem, rx_sem, (1 - x, y, z))
    rdx.start()
    rdx.wait_recv()

    cp1 = pltpu.make_async_copy(p_ref, va_ref, lsems.at[0])
    cp2 = pltpu.make_async_copy(pxr_ref, vb_ref, lsems.at[1])
    cp1.start()
    cp2.start()
    cp1.wait()
    cp2.wait()
    vs_ref[...] = (
        va_ref[...].astype(jnp.float32) + vb_ref[...].astype(jnp.float32)
    ).astype(jnp.bfloat16)
    rowA = gA * BLK
    rowB = gB * BLK
    st1 = pltpu.make_async_copy(
        vs_ref.at[:, pl.ds(0, HALF)], ha_ref.at[pl.ds(rowA, BLK)], lsems.at[0]
    )
    st2 = pltpu.make_async_copy(
        vs_ref.at[:, pl.ds(HALF, HALF)], hb_ref.at[pl.ds(rowB, BLK)], lsems.at[1]
    )
    st1.start()
    st2.start()
    st1.wait()
    st2.wait()
    rdx.wait_send()

    sent = []

    def cond_start(pred, desc):
        @pl.when(pred)
        def _():
            desc.start()

        sent.append((pred, desc))

    def cond_wait_recv(pred, desc):
        @pl.when(pred)
        def _():
            desc.wait_recv()

    def run_stage(flows):
        for f in flows:
            own = f["row"](f["pos"])
            sl = f["buf"].at[pl.ds(own, f["rows"])]
            cond_start(
                f["pos"] < 3,
                _remote(sl, sl, f["ssems"].at[0, f["pos"]], f["rsems"].at[0, f["pos"]], f["dev"](1)),
            )
            cond_start(
                f["pos"] > 0,
                _remote(sl, sl, f["ssems"].at[1, f["pos"]], f["rsems"].at[1, f["pos"]], f["dev"](-1)),
            )
        for h in (1, 2, 3):
            for f in flows:
                ql = f["pos"] - h
                qlc = jnp.clip(ql, 0, 3)
                sl = f["buf"].at[pl.ds(f["row"](qlc), f["rows"])]
                cond_wait_recv(
                    ql >= 0,
                    _remote(sl, sl, f["ssems"].at[0, qlc], f["rsems"].at[0, qlc], (x, y, z)),
                )
                cond_start(
                    (ql >= 0) & (f["pos"] < 3),
                    _remote(sl, sl, f["ssems"].at[0, qlc], f["rsems"].at[0, qlc], f["dev"](1)),
                )
                qr = f["pos"] + h
                qrc = jnp.clip(qr, 0, 3)
                sr = f["buf"].at[pl.ds(f["row"](qrc), f["rows"])]
                cond_wait_recv(
                    qr <= 3,
                    _remote(sr, sr, f["ssems"].at[1, qrc], f["rsems"].at[1, qrc], (x, y, z)),
                )
                cond_start(
                    (qr <= 3) & (f["pos"] > 0),
                    _remote(sr, sr, f["ssems"].at[1, qrc], f["rsems"].at[1, qrc], f["dev"](-1)),
                )

    def dev_y(d):
        return (x, jnp.clip(y + d, 0, 3), z)

    def dev_z(d):
        return (x, y, jnp.clip(z + d, 0, 3))

    run_stage(
        [
            dict(buf=ha_ref, pos=y, dev=dev_y, rows=BLK,
                 row=lambda q: (q + 4 * z) * BLK, ssems=sA1, rsems=rA1),
            dict(buf=hb_ref, pos=z, dev=dev_z, rows=BLK,
                 row=lambda q: (q + 4 * y) * BLK, ssems=sB1, rsems=rB1),
        ]
    )

    run_stage(
        [
            dict(buf=ha_ref, pos=z, dev=dev_z, rows=4 * BLK,
                 row=lambda q: q * 4 * BLK, ssems=sA2, rsems=rA2),
            dict(buf=hb_ref, pos=y, dev=dev_y, rows=4 * BLK,
                 row=lambda q: q * 4 * BLK, ssems=sB2, rsems=rB2),
        ]
    )

    for pred, desc in sent:
        @pl.when(pred)
        def _(desc=desc):
            desc.wait_send()

    for i in range(16):
        yi = i % 4
        zi = i // 4
        j = zi + 4 * yi
        la = pltpu.make_async_copy(
            ha_ref.at[pl.ds(i * BLK, BLK)], aa_ref, lsems.at[0]
        )
        lb = pltpu.make_async_copy(
            hb_ref.at[pl.ds(j * BLK, BLK)], ab_ref, lsems.at[1]
        )
        la.start()
        lb.start()
        la.wait()
        lb.wait()
        ao_ref[:, pl.ds(0, HALF)] = aa_ref[...].astype(jnp.float32)
        ao_ref[:, pl.ds(HALF, HALF)] = ab_ref[...].astype(jnp.float32)
        so = pltpu.make_async_copy(
            ao_ref, out_ref.at[pl.ds(i * BLK, BLK)], lsems.at[2]
        )
        so.start()
        so.wait()


def _comm(p_my):
    m = 16 * BLK
    out, _, _, _ = pl.pallas_call(
        _comm_body,
        in_specs=[pl.BlockSpec(memory_space=pl.ANY)],
        out_specs=[pl.BlockSpec(memory_space=pl.ANY)] * 4,
        out_shape=[
            jax.ShapeDtypeStruct((m, 2 * HALF), jnp.float32),
            jax.ShapeDtypeStruct((m, HALF), jnp.bfloat16),
            jax.ShapeDtypeStruct((m, HALF), jnp.bfloat16),
            jax.ShapeDtypeStruct((BLK, 2 * HALF), jnp.bfloat16),
        ],
        scratch_shapes=[
            pltpu.VMEM((BLK, 2 * HALF), jnp.bfloat16),
            pltpu.VMEM((BLK, 2 * HALF), jnp.bfloat16),
            pltpu.VMEM((BLK, 2 * HALF), jnp.bfloat16),
            pltpu.VMEM((BLK, HALF), jnp.bfloat16),
            pltpu.VMEM((BLK, HALF), jnp.bfloat16),
            pltpu.VMEM((BLK, 2 * HALF), jnp.float32),
            pltpu.SemaphoreType.DMA((4,)),
            pltpu.SemaphoreType.DMA,
            pltpu.SemaphoreType.DMA,
            pltpu.SemaphoreType.DMA((2, 4)),
            pltpu.SemaphoreType.DMA((2, 4)),
            pltpu.SemaphoreType.DMA((2, 4)),
            pltpu.SemaphoreType.DMA((2, 4)),
            pltpu.SemaphoreType.DMA((2, 4)),
            pltpu.SemaphoreType.DMA((2, 4)),
            pltpu.SemaphoreType.DMA((2, 4)),
            pltpu.SemaphoreType.DMA((2, 4)),
        ],
    )(p_my)
    return out


def kernel(dy, W):
    y = lax.axis_index("y")
    z = lax.axis_index("z")
    gA = y + 4 * z
    dy_blk = lax.dynamic_slice(dy, (gA * BLK, 0), (BLK, dy.shape[1]))
    p_my = _local_partial_block(dy_blk, W)
    return _comm(p_my)


# device time: 339528 ns/iter; 2.4880x vs baseline; 1.1360x over previous
import functools

import jax
import jax.numpy as jnp
from jax import lax
from jax.experimental import pallas as pl
from jax.experimental.pallas import tpu as pltpu

NY = 4
NZ = 4
BLK = 256
HALF = 2048
XS = 640


TN = 1024
TK = 2048


def _mm_body(dy_ref, w_ref, out_ref, acc_ref):
    k = pl.program_id(1)

    @pl.when(k == 0)
    def _():
        acc_ref[...] = jnp.zeros_like(acc_ref)

    a = dy_ref[:, pl.ds(k * TK, TK)].astype(jnp.bfloat16)
    b = w_ref[...].astype(jnp.bfloat16)
    acc_ref[...] += lax.dot_general(
        a, b, (((1,), (1,)), ((), ())), preferred_element_type=jnp.float32
    )

    @pl.when(k == pl.num_programs(1) - 1)
    def _():
        out_ref[...] = acc_ref[...].astype(jnp.bfloat16)


def _local_partial_block(dy_blk, w):
    m, kdim = dy_blk.shape
    n = w.shape[0]
    grid = (n // TN, kdim // TK)
    return pl.pallas_call(
        _mm_body,
        grid=grid,
        in_specs=[
            pl.BlockSpec((m, kdim), lambda j, k: (0, 0)),
            pl.BlockSpec((TN, TK), lambda j, k: (j, k)),
        ],
        out_specs=pl.BlockSpec((m, TN), lambda j, k: (0, j)),
        out_shape=jax.ShapeDtypeStruct((m, n), jnp.bfloat16),
        scratch_shapes=[pltpu.VMEM((m, TN), jnp.float32)],
        compiler_params=pltpu.CompilerParams(vmem_limit_bytes=100 * 1024 * 1024),
    )(dy_blk, w)


def _comm_body(
    p_ref,
    out_ref,
    ha_ref,
    hb_ref,
    pxr_ref,
    va_ref,
    vb_ref,
    vs_ref,
    aa_ref,
    ab_ref,
    ao_ref,
    lsems,
    sx_sem,
    rx_sem,
    sA1,
    rA1,
    sB1,
    rB1,
    sA2,
    rA2,
    sB2,
    rB2,
    sA2x,
    rA2x,
    sB2x,
    rB2x,
):
    x = lax.axis_index("x")
    y = lax.axis_index("y")
    z = lax.axis_index("z")
    gA = y + 4 * z
    gB = z + 4 * y

    def _remote(src, dst, ssem, rsem, dev):
        return pltpu.make_async_remote_copy(
            src_ref=src,
            dst_ref=dst,
            send_sem=ssem,
            recv_sem=rsem,
            device_id=dev,
            device_id_type=pl.DeviceIdType.MESH,
        )

    nbrs = [
        (None, (1 - x, y, z)),
        (y > 0, (x, jnp.clip(y - 1, 0, 3), z)),
        (y < 3, (x, jnp.clip(y + 1, 0, 3), z)),
        (z > 0, (x, y, jnp.clip(z - 1, 0, 3))),
        (z < 3, (x, y, jnp.clip(z + 1, 0, 3))),
    ]

    def _nbr_barrier(sem):
        for pred, dev in nbrs:
            if pred is None:
                pl.semaphore_signal(
                    sem, inc=1, device_id=dev, device_id_type=pl.DeviceIdType.MESH
                )
            else:
                @pl.when(pred)
                def _(dev=dev):
                    pl.semaphore_signal(
                        sem, inc=1, device_id=dev,
                        device_id_type=pl.DeviceIdType.MESH,
                    )
        for pred, _dev in nbrs:
            if pred is None:
                pl.semaphore_wait(sem, 1)
            else:
                @pl.when(pred)
                def _():
                    pl.semaphore_wait(sem, 1)

    _nbr_barrier(pltpu.get_barrier_semaphore())

    rdx = _remote(p_ref, pxr_ref, sx_sem, rx_sem, (1 - x, y, z))
    rdx.start()
    rdx.wait_recv()

    cp1 = pltpu.make_async_copy(p_ref, va_ref, lsems.at[0])
    cp2 = pltpu.make_async_copy(pxr_ref, vb_ref, lsems.at[1])
    cp1.start()
    cp2.start()
    cp1.wait()
    cp2.wait()
    vs_ref[...] = (
        va_ref[...].astype(jnp.float32) + vb_ref[...].astype(jnp.float32)
    ).astype(jnp.bfloat16)
    rowA = gA * BLK
    rowB = gB * BLK
    st1 = pltpu.make_async_copy(
        vs_ref.at[:, pl.ds(0, HALF)], ha_ref.at[pl.ds(rowA, BLK)], lsems.at[0]
    )
    st2 = pltpu.make_async_copy(
        vs_ref.at[:, pl.ds(HALF, HALF)], hb_ref.at[pl.ds(rowB, BLK)], lsems.at[1]
    )
    st1.start()
    st2.start()
    st1.wait()
    st2.wait()
    rdx.wait_send()

    sent = []

    def cond_start(pred, desc):
        @pl.when(pred)
        def _():
            desc.start()

        sent.append((pred, desc))

    def cond_wait_recv(pred, desc):
        @pl.when(pred)
        def _():
            desc.wait_recv()

    def run_stage(flows):
        def line_slice(f, q):
            row = f["row"](q)
            if f.get("xs"):
                return f["buf"].at[
                    pl.ds(row, f["rows"]), pl.ds(x * f["xs"], HALF - f["xs"])
                ]
            return f["buf"].at[pl.ds(row, f["rows"])]

        def xfwd(f, pred, q):
            src = f["buf"].at[
                pl.ds(f["row"](q), f["rows"]),
                pl.ds(x * (HALF - f["xs"]), f["xs"]),
            ]
            cond_start(
                pred,
                _remote(src, src, f["xss"].at[q], f["xrs"].at[q], (1 - x, y, z)),
            )

        for f in flows:
            sl = line_slice(f, f["pos"])
            cond_start(
                f["pos"] < 3,
                _remote(sl, sl, f["ssems"].at[0, f["pos"]], f["rsems"].at[0, f["pos"]], f["dev"](1)),
            )
            cond_start(
                f["pos"] > 0,
                _remote(sl, sl, f["ssems"].at[1, f["pos"]], f["rsems"].at[1, f["pos"]], f["dev"](-1)),
            )
        for h in (1, 2, 3):
            for f in flows:
                ql = f["pos"] - h
                qlc = jnp.clip(ql, 0, 3)
                sl = line_slice(f, qlc)
                cond_wait_recv(
                    ql >= 0,
                    _remote(sl, sl, f["ssems"].at[0, qlc], f["rsems"].at[0, qlc], (x, y, z)),
                )
                cond_start(
                    (ql >= 0) & (f["pos"] < 3),
                    _remote(sl, sl, f["ssems"].at[0, qlc], f["rsems"].at[0, qlc], f["dev"](1)),
                )
                if f.get("xs"):
                    xfwd(f, ql >= 0, qlc)
                qr = f["pos"] + h
                qrc = jnp.clip(qr, 0, 3)
                sr = line_slice(f, qrc)
                cond_wait_recv(
                    qr <= 3,
                    _remote(sr, sr, f["ssems"].at[1, qrc], f["rsems"].at[1, qrc], (x, y, z)),
                )
                cond_start(
                    (qr <= 3) & (f["pos"] > 0),
                    _remote(sr, sr, f["ssems"].at[1, qrc], f["rsems"].at[1, qrc], f["dev"](-1)),
                )
                if f.get("xs"):
                    xfwd(f, qr <= 3, qrc)
        for h in (1, 2, 3):
            for f in flows:
                if not f.get("xs"):
                    continue
                for q, pred in ((f["pos"] - h, f["pos"] - h >= 0),
                                (f["pos"] + h, f["pos"] + h <= 3)):
                    qc = jnp.clip(q, 0, 3)
                    dst = f["buf"].at[
                        pl.ds(f["row"](qc), f["rows"]),
                        pl.ds((1 - x) * (HALF - f["xs"]), f["xs"]),
                    ]
                    cond_wait_recv(
                        pred,
                        _remote(dst, dst, f["xss"].at[qc], f["xrs"].at[qc], (x, y, z)),
                    )

    def dev_y(d):
        return (x, jnp.clip(y + d, 0, 3), z)

    def dev_z(d):
        return (x, y, jnp.clip(z + d, 0, 3))

    run_stage(
        [
            dict(buf=ha_ref, pos=y, dev=dev_y, rows=BLK,
                 row=lambda q: (q + 4 * z) * BLK, ssems=sA1, rsems=rA1),
            dict(buf=hb_ref, pos=z, dev=dev_z, rows=BLK,
                 row=lambda q: (q + 4 * y) * BLK, ssems=sB1, rsems=rB1),
        ]
    )

    run_stage(
        [
            dict(buf=ha_ref, pos=z, dev=dev_z, rows=4 * BLK,
                 row=lambda q: q * 4 * BLK, ssems=sA2, rsems=rA2,
                 xs=XS, xss=sA2x, xrs=rA2x),
            dict(buf=hb_ref, pos=y, dev=dev_y, rows=4 * BLK,
                 row=lambda q: q * 4 * BLK, ssems=sB2, rsems=rB2,
                 xs=XS, xss=sB2x, xrs=rB2x),
        ]
    )

    for pred, desc in sent:
        @pl.when(pred)
        def _(desc=desc):
            desc.wait_send()

    def _loads(i):
        s = i % 2
        j = (i // 4) + 4 * (i % 4)
        da = pltpu.make_async_copy(
            ha_ref.at[pl.ds(i * BLK, BLK)], aa_ref.at[s], lsems.at[s]
        )
        db = pltpu.make_async_copy(
            hb_ref.at[pl.ds(j * BLK, BLK)], ab_ref.at[s], lsems.at[2 + s]
        )
        da.start()
        db.start()
        return da, db

    loads = {0: _loads(0)}
    stores = {}
    for i in range(16):
        s = i % 2
        if i < 15:
            loads[i + 1] = _loads(i + 1)
        da, db = loads.pop(i)
        da.wait()
        db.wait()
        if i - 2 in stores:
            stores.pop(i - 2).wait()
        ao_ref[s, :, pl.ds(0, HALF)] = aa_ref[s].astype(jnp.float32)
        ao_ref[s, :, pl.ds(HALF, HALF)] = ab_ref[s].astype(jnp.float32)
        so = pltpu.make_async_copy(
            ao_ref.at[s], out_ref.at[pl.ds(i * BLK, BLK)], lsems.at[4 + s]
        )
        so.start()
        stores[i] = so
    for i in (14, 15):
        stores.pop(i).wait()

    @functools.partial(pl.run_scoped, sem2=pltpu.SemaphoreType.REGULAR)
    def _(sem2):
        _nbr_barrier(sem2)


def _comm(p_my):
    m = 16 * BLK
    out, _, _, _ = pl.pallas_call(
        _comm_body,
        in_specs=[pl.BlockSpec(memory_space=pl.ANY)],
        out_specs=[pl.BlockSpec(memory_space=pl.ANY)] * 4,
        out_shape=[
            jax.ShapeDtypeStruct((m, 2 * HALF), jnp.float32),
            jax.ShapeDtypeStruct((m, HALF), jnp.bfloat16),
            jax.ShapeDtypeStruct((m, HALF), jnp.bfloat16),
            jax.ShapeDtypeStruct((BLK, 2 * HALF), jnp.bfloat16),
        ],
        scratch_shapes=[
            pltpu.VMEM((BLK, 2 * HALF), jnp.bfloat16),
            pltpu.VMEM((BLK, 2 * HALF), jnp.bfloat16),
            pltpu.VMEM((BLK, 2 * HALF), jnp.bfloat16),
            pltpu.VMEM((2, BLK, HALF), jnp.bfloat16),
            pltpu.VMEM((2, BLK, HALF), jnp.bfloat16),
            pltpu.VMEM((2, BLK, 2 * HALF), jnp.float32),
            pltpu.SemaphoreType.DMA((6,)),
            pltpu.SemaphoreType.DMA,
            pltpu.SemaphoreType.DMA,
            pltpu.SemaphoreType.DMA((2, 4)),
            pltpu.SemaphoreType.DMA((2, 4)),
            pltpu.SemaphoreType.DMA((2, 4)),
            pltpu.SemaphoreType.DMA((2, 4)),
            pltpu.SemaphoreType.DMA((2, 4)),
            pltpu.SemaphoreType.DMA((2, 4)),
            pltpu.SemaphoreType.DMA((2, 4)),
            pltpu.SemaphoreType.DMA((2, 4)),
            pltpu.SemaphoreType.DMA((4,)),
            pltpu.SemaphoreType.DMA((4,)),
            pltpu.SemaphoreType.DMA((4,)),
            pltpu.SemaphoreType.DMA((4,)),
        ],
        compiler_params=pltpu.CompilerParams(collective_id=0),
    )(p_my)
    return out


def kernel(dy, W):
    y = lax.axis_index("y")
    z = lax.axis_index("z")
    gA = y + 4 * z
    dy_blk = lax.dynamic_slice(dy, (gA * BLK, 0), (BLK, dy.shape[1]))
    p_my = _local_partial_block(dy_blk, W)
    return _comm(p_my)
